# baseline (speedup 1.0000x reference)
import functools

import jax
import jax.numpy as jnp
from jax import lax
from jax.experimental import pallas as pl
from jax.experimental.pallas import tpu as pltpu
from jax.experimental.pallas import tpu_sc as plsc

B = 8
L = 1024
D = 32
V = 8192
BLK = 1024
NBLK = V // BLK
COMMIT = 0.02
EPS = 1e-12

_SC_CORES = 2
_SC_SUBCORES = 16
_SC_WORKERS = _SC_CORES * _SC_SUBCORES
_ROWS_PER_WORKER = (B * L) // _SC_WORKERS


def _vq_body(x_ref, w_ref, idx_ref, loss_ref, bd, bi, bw, lacc):
    b = pl.program_id(0)
    nb = pl.program_id(1)

    xb = x_ref[0]
    w = w_ref[...]

    wnsq = jnp.sum(w * w, axis=1, keepdims=True)
    wn = w / jnp.maximum(jnp.sqrt(wnsq), EPS)
    xnsq = jnp.sum(xb * xb, axis=0, keepdims=True)
    nx = jnp.maximum(jnp.sqrt(xnsq), EPS)
    xn = xb / nx

    s = lax.dot_general(wn, xn, (((1,), (0,)), ((), ())),
                        preferred_element_type=jnp.float32)
    d = 2.0 - 2.0 * s

    m = jnp.min(d, axis=0, keepdims=True)
    rio = lax.broadcasted_iota(jnp.int32, d.shape, 0)
    hit = d == m
    li = jnp.min(jnp.where(hit, rio, jnp.int32(2**31 - 1)),
                 axis=0, keepdims=True)
    oh = rio == li
    wsel = jnp.sum(jnp.where(oh, wnsq, 0.0), axis=0, keepdims=True)

    @pl.when(nb == 0)
    def _():
        bd[...] = jnp.full((1, L), jnp.inf, jnp.float32)
        bi[...] = jnp.zeros((1, L), jnp.int32)
        bw[...] = jnp.zeros((1, L), jnp.float32)

    upd = m < bd[...]
    bd[...] = jnp.where(upd, m, bd[...])
    bi[...] = jnp.where(upd, nb * BLK + li, bi[...])
    bw[...] = jnp.where(upd, wsel, bw[...])

    @pl.when(nb == NBLK - 1)
    def _():
        idx_ref[0, 0, :] = bi[0, :]
        nw = jnp.maximum(jnp.sqrt(bw[...]), EPS)
        qx = nx * nw * (2.0 - bd[...]) * 0.5
        contrib = bw[...] - 2.0 * qx + xnsq
        psum = jnp.sum(contrib)

        @pl.when(b == 0)
        def _():
            lacc[0] = 0.0

        lacc[0] += psum

        @pl.when(b == B - 1)
        def _():
            loss_ref[0, 0] = lacc[0] * ((1.0 + COMMIT) / (B * L * D))


_vq_argmin_call = pl.pallas_call(
    _vq_body,
    grid=(B, NBLK),
    in_specs=[
        pl.BlockSpec((1, D, L), lambda b, nb: (b, 0, 0)),
        pl.BlockSpec((BLK, D), lambda b, nb: (nb, 0)),
    ],
    out_specs=[
        pl.BlockSpec((1, 1, L), lambda b, nb: (b, 0, 0)),
        pl.BlockSpec(memory_space=pltpu.SMEM),
    ],
    out_shape=[
        jax.ShapeDtypeStruct((B, 1, L), jnp.int32),
        jax.ShapeDtypeStruct((1, 1), jnp.float32),
    ],
    scratch_shapes=[
        pltpu.VMEM((1, L), jnp.float32),
        pltpu.VMEM((1, L), jnp.int32),
        pltpu.VMEM((1, L), jnp.float32),
        pltpu.SMEM((1,), jnp.float32),
    ],
    compiler_params=pltpu.CompilerParams(
        dimension_semantics=("arbitrary", "arbitrary")),
)


@functools.cache
def _make_sc_gather():
    @functools.partial(
        pl.kernel,
        mesh=plsc.VectorSubcoreMesh(core_axis_name="c", subcore_axis_name="s"),
        out_type=jax.ShapeDtypeStruct((B * L, D), jnp.float32),
        scratch_types=[
            pltpu.VMEM((_ROWS_PER_WORKER,), jnp.int32),
            pltpu.VMEM((_ROWS_PER_WORKER, D), jnp.float32),
            pltpu.SemaphoreType.DMA,
        ],
        compiler_params=pltpu.CompilerParams(use_tc_tiling_on_sc=False),
    )
    def _sc_gather(w_hbm, idx_hbm, out_hbm, idx_v, rows_v, sem):
        wid = lax.axis_index("s") * _SC_CORES + lax.axis_index("c")
        base = wid * _ROWS_PER_WORKER
        pltpu.sync_copy(idx_hbm.at[pl.ds(base, _ROWS_PER_WORKER)], idx_v)
        pltpu.async_copy(w_hbm.at[idx_v], rows_v, sem).wait()
        pltpu.sync_copy(rows_v, out_hbm.at[pl.ds(base, _ROWS_PER_WORKER)])

    return _sc_gather


def kernel(inputs, weight):
    idx_blk, loss11 = _vq_argmin_call(inputs, weight)
    idx_flat = idx_blk.reshape(B * L)
    q = _make_sc_gather()(weight, idx_flat)
    quantized_out = q.reshape(B, L, D).transpose(0, 2, 1)
    loss = loss11[0, 0]
    encoding_indices = idx_flat.reshape(B * L, 1)
    return (loss, quantized_out, encoding_indices)


# speedup vs baseline: 1.2614x; 1.2614x over previous
import functools

import jax
import jax.numpy as jnp
from jax import lax
from jax.experimental import pallas as pl
from jax.experimental.pallas import tpu as pltpu
from jax.experimental.pallas import tpu_sc as plsc

B = 8
L = 1024
D = 32
V = 8192
BLK = 1024
NBLK = V // BLK
COMMIT = 0.02
EPS = 1e-12
IBIG = 2**31 - 1

_SC_CORES = 2
_SC_SUBCORES = 16
_SC_WORKERS = _SC_CORES * _SC_SUBCORES
_ROWS_PER_WORKER = (B * L) // _SC_WORKERS


def _vq_body(x_ref, w_ref, idx_ref, bd_ref, xns_ref,
             bs, bi, rio, wn_s, xn_s):
    b = pl.program_id(0)
    nb = pl.program_id(1)

    @pl.when(jnp.logical_and(b == 0, nb == 0))
    def _():
        rio[...] = lax.broadcasted_iota(jnp.int32, (BLK, L), 0)

    @pl.when(b == 0)
    def _():
        w = w_ref[...]
        wnsq = jnp.sum(w * w, axis=1, keepdims=True)
        wn_s[pl.ds(nb * BLK, BLK), :] = w / jnp.maximum(jnp.sqrt(wnsq), EPS)

    @pl.when(nb == 0)
    def _():
        xb = x_ref[0]
        xnsq = jnp.sum(xb * xb, axis=0, keepdims=True)
        xns_ref[0] = xnsq
        xn_s[...] = xb / jnp.maximum(jnp.sqrt(xnsq), EPS)

    wn = wn_s[pl.ds(nb * BLK, BLK), :]
    s = lax.dot_general(wn, xn_s[...], (((1,), (0,)), ((), ())),
                        preferred_element_type=jnp.float32)

    smax = jnp.max(s, axis=0, keepdims=True)
    hit = s == smax
    li = jnp.min(jnp.where(hit, rio[...], IBIG),
                 axis=0, keepdims=True)

    @pl.when(nb == 0)
    def _():
        bs[...] = jnp.full((1, L), -jnp.inf, jnp.float32)
        bi[...] = jnp.zeros((1, L), jnp.int32)

    upd = smax > bs[...]
    bs[...] = jnp.where(upd, smax, bs[...])
    bi[...] = jnp.where(upd, nb * BLK + li, bi[...])

    @pl.when(nb == NBLK - 1)
    def _():
        idx_ref[0, 0, :] = bi[0, :]
        bd_ref[0, 0, :] = 2.0 - 2.0 * bs[0, :]


_vq_argmin_call = pl.pallas_call(
    _vq_body,
    grid=(B, NBLK),
    in_specs=[
        pl.BlockSpec((1, D, L), lambda b, nb: (b, 0, 0)),
        pl.BlockSpec((BLK, D), lambda b, nb: (nb, 0)),
    ],
    out_specs=[
        pl.BlockSpec((1, 1, L), lambda b, nb: (b, 0, 0)),
        pl.BlockSpec((1, 1, L), lambda b, nb: (b, 0, 0)),
        pl.BlockSpec((1, 1, L), lambda b, nb: (b, 0, 0)),
    ],
    out_shape=[
        jax.ShapeDtypeStruct((B, 1, L), jnp.int32),
        jax.ShapeDtypeStruct((B, 1, L), jnp.float32),
        jax.ShapeDtypeStruct((B, 1, L), jnp.float32),
    ],
    scratch_shapes=[
        pltpu.VMEM((1, L), jnp.float32),
        pltpu.VMEM((1, L), jnp.int32),
        pltpu.VMEM((BLK, L), jnp.int32),
        pltpu.VMEM((V, D), jnp.float32),
        pltpu.VMEM((D, L), jnp.float32),
    ],
    compiler_params=pltpu.CompilerParams(
        dimension_semantics=("arbitrary", "arbitrary")),
)


def _loss_body(q_ref, bd_ref, xns_ref, loss_ref, lacc):
    b = pl.program_id(0)
    qb = q_ref[...]
    qnsq = jnp.sum(qb * qb, axis=1, keepdims=True)
    nw = jnp.maximum(jnp.sqrt(qnsq), EPS)
    bd = bd_ref[0]
    xns = xns_ref[0]
    nx = jnp.maximum(jnp.sqrt(xns), EPS)
    crossvec = nx * (2.0 - bd) * 0.5
    cross = lax.dot_general(crossvec, nw, (((1,), (0,)), ((), ())),
                            preferred_element_type=jnp.float32)
    total = jnp.sum(qnsq) - 2.0 * cross[0, 0] + jnp.sum(xns)

    @pl.when(b == 0)
    def _():
        lacc[0] = 0.0

    lacc[0] += total

    @pl.when(b == B - 1)
    def _():
        loss_ref[0, 0] = lacc[0] * ((1.0 + COMMIT) / (B * L * D))


_loss_call = pl.pallas_call(
    _loss_body,
    grid=(B,),
    in_specs=[
        pl.BlockSpec((L, D), lambda b: (b, 0)),
        pl.BlockSpec((1, 1, L), lambda b: (b, 0, 0)),
        pl.BlockSpec((1, 1, L), lambda b: (b, 0, 0)),
    ],
    out_specs=pl.BlockSpec(memory_space=pltpu.SMEM),
    out_shape=jax.ShapeDtypeStruct((1, 1), jnp.float32),
    scratch_shapes=[pltpu.SMEM((1,), jnp.float32)],
    compiler_params=pltpu.CompilerParams(
        dimension_semantics=("arbitrary",)),
)


@functools.cache
def _make_sc_gather():
    @functools.partial(
        pl.kernel,
        mesh=plsc.VectorSubcoreMesh(core_axis_name="c", subcore_axis_name="s"),
        out_type=jax.ShapeDtypeStruct((B * L, D), jnp.float32),
        scratch_types=[
            pltpu.VMEM((_ROWS_PER_WORKER,), jnp.int32),
            pltpu.VMEM((_ROWS_PER_WORKER, D), jnp.float32),
            pltpu.SemaphoreType.DMA,
        ],
        compiler_params=pltpu.CompilerParams(use_tc_tiling_on_sc=False),
    )
    def _sc_gather(w_hbm, idx_hbm, out_hbm, idx_v, rows_v, sem):
        wid = lax.axis_index("s") * _SC_CORES + lax.axis_index("c")
        base = wid * _ROWS_PER_WORKER
        pltpu.sync_copy(idx_hbm.at[pl.ds(base, _ROWS_PER_WORKER)], idx_v)
        pltpu.async_copy(w_hbm.at[idx_v], rows_v, sem).wait()
        pltpu.sync_copy(rows_v, out_hbm.at[pl.ds(base, _ROWS_PER_WORKER)])

    return _sc_gather


def kernel(inputs, weight):
    idx_blk, bd, xns = _vq_argmin_call(inputs, weight)
    idx_flat = idx_blk.reshape(B * L)
    q = _make_sc_gather()(weight, idx_flat)
    loss11 = _loss_call(q, bd, xns)
    quantized_out = q.reshape(B, L, D).transpose(0, 2, 1)
    loss = loss11[0, 0]
    encoding_indices = idx_flat.reshape(B * L, 1)
    return (loss, quantized_out, encoding_indices)


# speedup vs baseline: 1.4991x; 1.1885x over previous
import functools

import jax
import jax.numpy as jnp
from jax import lax
from jax.experimental import pallas as pl
from jax.experimental.pallas import tpu as pltpu
from jax.experimental.pallas import tpu_sc as plsc

B = 8
L = 1024
N = B * L
D = 32
V = 8192
BLK = 1024
NBLK = V // BLK
LB = 4096
NL = N // LB
COMMIT = 0.02
EPS = 1e-12

_SC_CORES = 2
_SC_SUBCORES = 16
_SC_WORKERS = _SC_CORES * _SC_SUBCORES
_ROWS_PER_WORKER = N // _SC_WORKERS


def _prep_body(x_ref, w_ref, xn_ref, xns_ref, wn_ref):
    x = x_ref[...]
    xnsq = jnp.sum(x * x, axis=0, keepdims=True)
    xns_ref[...] = xnsq
    xn_ref[...] = x / jnp.maximum(jnp.sqrt(xnsq), EPS)
    w = w_ref[...]
    wnsq = jnp.sum(w * w, axis=1, keepdims=True)
    wn_ref[...] = w / jnp.maximum(jnp.sqrt(wnsq), EPS)


_prep_call = pl.pallas_call(
    _prep_body,
    out_shape=[
        jax.ShapeDtypeStruct((D, N), jnp.float32),
        jax.ShapeDtypeStruct((1, N), jnp.float32),
        jax.ShapeDtypeStruct((V, D), jnp.float32),
    ],
)


def _vq_body(xn_ref, wn_ref, idx_ref, bd_ref, bs, bi, rio):
    nl = pl.program_id(0)
    nb = pl.program_id(1)

    @pl.when(jnp.logical_and(nl == 0, nb == 0))
    def _():
        rio[...] = lax.broadcasted_iota(
            jnp.int32, (BLK, LB), 0).astype(jnp.float32)

    s = lax.dot_general(wn_ref[...], xn_ref[...], (((1,), (0,)), ((), ())),
                        preferred_element_type=jnp.float32)

    smax = jnp.max(s, axis=0, keepdims=True)
    hit = s == smax
    li = jnp.min(jnp.where(hit, rio[...], 1e9),
                 axis=0, keepdims=True)

    @pl.when(nb == 0)
    def _():
        bs[...] = jnp.full((1, LB), -jnp.inf, jnp.float32)
        bi[...] = jnp.zeros((1, LB), jnp.float32)

    upd = smax > bs[...]
    bs[...] = jnp.where(upd, smax, bs[...])
    bi[...] = jnp.where(upd, float(BLK) * nb + li, bi[...])

    @pl.when(nb == NBLK - 1)
    def _():
        idx_ref[...] = bi[...].astype(jnp.int32)
        bd_ref[...] = 2.0 - 2.0 * bs[...]


_vq_argmin_call = pl.pallas_call(
    _vq_body,
    grid=(NL, NBLK),
    in_specs=[
        pl.BlockSpec((D, LB), lambda nl, nb: (0, nl)),
        pl.BlockSpec((BLK, D), lambda nl, nb: (nb, 0)),
    ],
    out_specs=[
        pl.BlockSpec((1, LB), lambda nl, nb: (0, nl)),
        pl.BlockSpec((1, LB), lambda nl, nb: (0, nl)),
    ],
    out_shape=[
        jax.ShapeDtypeStruct((1, N), jnp.int32),
        jax.ShapeDtypeStruct((1, N), jnp.float32),
    ],
    scratch_shapes=[
        pltpu.VMEM((1, LB), jnp.float32),
        pltpu.VMEM((1, LB), jnp.float32),
        pltpu.VMEM((BLK, LB), jnp.float32),
    ],
    compiler_params=pltpu.CompilerParams(
        dimension_semantics=("arbitrary", "arbitrary")),
)


def _loss_body(q_ref, bd_ref, xns_ref, loss_ref, lacc):
    b = pl.program_id(0)
    qb = q_ref[...]
    qnsq = jnp.sum(qb * qb, axis=1, keepdims=True)
    nw = jnp.maximum(jnp.sqrt(qnsq), EPS)
    bd = bd_ref[...]
    xns = xns_ref[...]
    nx = jnp.maximum(jnp.sqrt(xns), EPS)
    crossvec = nx * (2.0 - bd) * 0.5
    cross = lax.dot_general(crossvec, nw, (((1,), (0,)), ((), ())),
                            preferred_element_type=jnp.float32)
    total = jnp.sum(qnsq) - 2.0 * cross[0, 0] + jnp.sum(xns)

    @pl.when(b == 0)
    def _():
        lacc[0] = 0.0

    lacc[0] += total

    @pl.when(b == B - 1)
    def _():
        loss_ref[0, 0] = lacc[0] * ((1.0 + COMMIT) / (N * D))


_loss_call = pl.pallas_call(
    _loss_body,
    grid=(B,),
    in_specs=[
        pl.BlockSpec((L, D), lambda b: (b, 0)),
        pl.BlockSpec((1, L), lambda b: (0, b)),
        pl.BlockSpec((1, L), lambda b: (0, b)),
    ],
    out_specs=pl.BlockSpec(memory_space=pltpu.SMEM),
    out_shape=jax.ShapeDtypeStruct((1, 1), jnp.float32),
    scratch_shapes=[pltpu.SMEM((1,), jnp.float32)],
    compiler_params=pltpu.CompilerParams(
        dimension_semantics=("arbitrary",)),
)


@functools.cache
def _make_sc_gather():
    @functools.partial(
        pl.kernel,
        mesh=plsc.VectorSubcoreMesh(core_axis_name="c", subcore_axis_name="s"),
        out_type=jax.ShapeDtypeStruct((N, D), jnp.float32),
        scratch_types=[
            pltpu.VMEM((_ROWS_PER_WORKER,), jnp.int32),
            pltpu.VMEM((_ROWS_PER_WORKER, D), jnp.float32),
            pltpu.SemaphoreType.DMA,
        ],
        compiler_params=pltpu.CompilerParams(use_tc_tiling_on_sc=False),
    )
    def _sc_gather(w_hbm, idx_hbm, out_hbm, idx_v, rows_v, sem):
        wid = lax.axis_index("s") * _SC_CORES + lax.axis_index("c")
        base = wid * _ROWS_PER_WORKER
        pltpu.sync_copy(idx_hbm.at[pl.ds(base, _ROWS_PER_WORKER)], idx_v)
        pltpu.async_copy(w_hbm.at[idx_v], rows_v, sem).wait()
        pltpu.sync_copy(rows_v, out_hbm.at[pl.ds(base, _ROWS_PER_WORKER)])

    return _sc_gather


def kernel(inputs, weight):
    x2 = inputs.transpose(1, 0, 2).reshape(D, N)
    xn, xns, wn = _prep_call(x2, weight)
    idx_row, bd = _vq_argmin_call(xn, wn)
    idx_flat = idx_row.reshape(N)
    q = _make_sc_gather()(weight, idx_flat)
    loss11 = _loss_call(q, bd, xns)
    quantized_out = q.reshape(B, L, D).transpose(0, 2, 1)
    loss = loss11[0, 0]
    encoding_indices = idx_flat.reshape(N, 1)
    return (loss, quantized_out, encoding_indices)
